# baseline (device time: 83706 ns/iter reference)
import jax
import jax.numpy as jnp
from jax import lax
from jax.experimental import pallas as pl
from jax.experimental.pallas import tpu as pltpu

N_DEV = 8
B_LOC = 2
SQ = 128
SKV = 128
HQ = 32
DH = 64
H_LOC = HQ // N_DEV
D_MODEL = 512
HD_LOC = H_LOC * DH
SBLK = H_LOC * SKV
VAUG = HD_LOC + 128

CW_HOPS = 4
CCW_HOPS = 3

LOCAL = 7


def kernel(x, Wq, K_ext, V_ext, Wo):
    def body(x_ref, wq_ref, wo_ref, k_ref, v_ref, out_ref,
             g_qwq, g_sq, g_qwo, g_so, xv, wqv, wov, kst, vst,
             xb, ctx_scr, kblkT, vblk,
             sem_in, sem_k, sem_v,
             s_qwq, r_qwq, s_sq, r_sq, s_qwo, r_qwo, s_so, r_so):
        my_pos = lax.axis_index("i")
        left = lax.rem(my_pos + N_DEV - 1, N_DEV)
        right = lax.rem(my_pos + 1, N_DEV)

        barrier = pltpu.get_barrier_semaphore()
        for nbr in (left, right):
            pl.semaphore_signal(barrier, inc=1, device_id=(nbr,),
                                device_id_type=pl.DeviceIdType.MESH)
        pl.semaphore_wait(barrier, 2)

        cx = pltpu.make_async_copy(x_ref, xv, sem_in.at[0])
        cwq = pltpu.make_async_copy(wq_ref, wqv, sem_in.at[1])
        cwo = pltpu.make_async_copy(wo_ref, wov, sem_in.at[2])
        for c in (cx, cwq, cwo):
            c.start()

        staged = {}

        def stage(d):
            grp = lax.rem(my_pos - d + N_DEV, N_DEV)
            g4 = grp * H_LOC
            cps = []
            for b in range(B_LOC):
                bidx = my_pos * B_LOC + b
                for hh in range(H_LOC):
                    cps.append(pltpu.make_async_copy(
                        k_ref.at[bidx, :, g4 + hh, :],
                        kst.at[d, b, hh], sem_k.at[d]))
                    cps.append(pltpu.make_async_copy(
                        v_ref.at[bidx, :, g4 + hh, :],
                        vst.at[d, b, hh], sem_v.at[d]))
            for c in cps:
                c.start()
            staged[d] = cps

        def wait_stage(d):
            for c in staged[d]:
                c.wait()

        stage(0)

        cwq.wait()
        wq = wqv[...]
        aq = jnp.maximum(jnp.max(jnp.abs(wq), axis=0, keepdims=True), 1e-30)
        g_qwq[LOCAL] = jnp.round(wq * (127.0 / aq)).astype(jnp.int8)
        g_sq[LOCAL] = aq * (0.125 / 127.0)
        cwo.wait()
        wo = wov[...]
        ao = jnp.maximum(jnp.max(jnp.abs(wo), axis=0, keepdims=True), 1e-30)
        g_qwo[LOCAL] = jnp.round(wo * (127.0 / ao)).astype(jnp.int8)
        g_so[LOCAL] = ao * (1.0 / 127.0)

        def chunk_rdmas(src_slot, dst_slot, sem_idx, target):
            descs = []
            for g, ssem, rsem in ((g_qwq, s_qwq, r_qwq),
                                  (g_sq, s_sq, r_sq),
                                  (g_qwo, s_qwo, r_qwo),
                                  (g_so, s_so, r_so)):
                descs.append(pltpu.make_async_remote_copy(
                    src_ref=g.at[src_slot], dst_ref=g.at[dst_slot],
                    send_sem=ssem.at[sem_idx], recv_sem=rsem.at[dst_slot],
                    device_id=(target,), device_id_type=pl.DeviceIdType.MESH))
            return descs

        def send_chunk(src_slot, dst_slot, sem_idx, target):
            descs = chunk_rdmas(src_slot, dst_slot, sem_idx, target)
            for d_ in descs:
                d_.start()
            return descs

        def wait_recv(slot):
            for d_ in chunk_rdmas(LOCAL, slot, 0, left):
                d_.wait_recv()

        sends = []
        sends += send_chunk(LOCAL, 0, 0, right)
        sends += send_chunk(LOCAL, 6, 4, left)

        stage(1)
        stage(7)

        for b in range(B_LOC):
            kblkT[b] = jnp.zeros((SBLK, HD_LOC), jnp.bfloat16)
            rows = lax.broadcasted_iota(jnp.int32, (SBLK, 128), 0) // SKV
            cols = lax.broadcasted_iota(jnp.int32, (SBLK, 128), 1)
            ones_pat = (rows == cols).astype(jnp.bfloat16)
            vblk[b] = jnp.concatenate(
                [jnp.zeros((SBLK, HD_LOC), jnp.bfloat16), ones_pat], axis=1)

        cx.wait()
        for b in range(B_LOC):
            xb[b * SQ:(b + 1) * SQ, :] = xv[b].astype(jnp.bfloat16)

        qblk = lax.broadcasted_iota(jnp.int32, (SQ, SBLK), 0) // 64
        kblk_id = (lax.broadcasted_iota(jnp.int32, (SQ, SBLK), 1) % SKV) // 64
        mask = kblk_id <= qblk

        def compute(d, slot, first):
            wait_stage(d)
            q_all = (jax.lax.dot_general(
                xb[...], g_qwq[slot].astype(jnp.bfloat16),
                (((1,), (0,)), ((), ())),
                preferred_element_type=jnp.float32,
            ) * g_sq[slot]).astype(jnp.bfloat16)
            for b in range(B_LOC):
                for hh in range(H_LOC):
                    kblkT[b, hh * SKV:(hh + 1) * SKV,
                          hh * DH:(hh + 1) * DH] = (
                        kst[d, b, hh].astype(jnp.bfloat16))
                    vblk[b, hh * SKV:(hh + 1) * SKV,
                         hh * DH:(hh + 1) * DH] = (
                        vst[d, b, hh].astype(jnp.bfloat16))
            for b in range(B_LOC):
                s = jax.lax.dot_general(
                    q_all[b * SQ:(b + 1) * SQ, :], kblkT[b],
                    (((1,), (1,)), ((), ())),
                    preferred_element_type=jnp.float32)
                w = jnp.where(mask, jnp.exp(s), 0.0).astype(jnp.bfloat16)
                aug = jax.lax.dot_general(
                    w, vblk[b],
                    (((1,), (0,)), ((), ())),
                    preferred_element_type=jnp.float32)
                rec = 1.0 / aug[:, HD_LOC:HD_LOC + H_LOC]
                scale = jnp.broadcast_to(
                    rec[:, :, None], (SQ, H_LOC, DH)).reshape(SQ, HD_LOC)
                ctx_scr[b * SQ:(b + 1) * SQ, :] = (
                    aug[:, :HD_LOC] * scale).astype(jnp.bfloat16)
            contrib = jax.lax.dot_general(
                ctx_scr[...], g_qwo[slot].astype(jnp.bfloat16),
                (((1,), (0,)), ((), ())),
                preferred_element_type=jnp.float32) * g_so[slot]
            if first:
                out_ref[...] = contrib
            else:
                out_ref[...] = out_ref[...] + contrib

        compute(0, LOCAL, first=True)

        for r in range(1, 4):
            cw = r - 1
            ccw = 7 - r
            wait_recv(cw)
            if r < CW_HOPS:
                sends += send_chunk(cw, cw + 1, r, right)
            wait_recv(ccw)
            if r < CCW_HOPS:
                sends += send_chunk(ccw, ccw - 1, 4 + r, left)
            if r + 1 <= 4:
                stage(r + 1)
            if 8 - r - 1 >= 5:
                stage(8 - r - 1)
            compute(r, cw, first=False)
            compute(8 - r, ccw, first=False)

        wait_recv(3)
        compute(4, 3, first=False)

        for s_ in sends:
            s_.wait_send()

    out = pl.pallas_call(
        body,
        out_shape=jax.ShapeDtypeStruct((B_LOC * SQ, D_MODEL), jnp.float32),
        in_specs=[pl.BlockSpec(memory_space=pl.ANY)] * 5,
        out_specs=pl.BlockSpec(memory_space=pltpu.VMEM),
        scratch_shapes=[
            pltpu.VMEM((8, D_MODEL, HD_LOC), jnp.int8),
            pltpu.VMEM((8, 1, HD_LOC), jnp.float32),
            pltpu.VMEM((8, HD_LOC, D_MODEL), jnp.int8),
            pltpu.VMEM((8, 1, D_MODEL), jnp.float32),
            pltpu.VMEM((B_LOC, SQ, D_MODEL), jnp.float32),
            pltpu.VMEM((D_MODEL, HD_LOC), jnp.float32),
            pltpu.VMEM((HD_LOC, D_MODEL), jnp.float32),
            pltpu.VMEM((8, B_LOC, H_LOC, SKV, DH), jnp.float32),
            pltpu.VMEM((8, B_LOC, H_LOC, SKV, DH), jnp.float32),
            pltpu.VMEM((B_LOC * SQ, D_MODEL), jnp.bfloat16),
            pltpu.VMEM((B_LOC * SQ, HD_LOC), jnp.bfloat16),
            pltpu.VMEM((B_LOC, SBLK, HD_LOC), jnp.bfloat16),
            pltpu.VMEM((B_LOC, SBLK, VAUG), jnp.bfloat16),
            pltpu.SemaphoreType.DMA((3,)),
            pltpu.SemaphoreType.DMA((8,)),
            pltpu.SemaphoreType.DMA((8,)),
            pltpu.SemaphoreType.DMA((8,)),
            pltpu.SemaphoreType.DMA((8,)),
            pltpu.SemaphoreType.DMA((8,)),
            pltpu.SemaphoreType.DMA((8,)),
            pltpu.SemaphoreType.DMA((8,)),
            pltpu.SemaphoreType.DMA((8,)),
            pltpu.SemaphoreType.DMA((8,)),
            pltpu.SemaphoreType.DMA((8,)),
        ],
        compiler_params=pltpu.CompilerParams(collective_id=0),
    )(x, Wq, Wo, K_ext, V_ext)

    return out.reshape(B_LOC, SQ, D_MODEL)


# device time: 32139 ns/iter; 2.6045x vs baseline; 2.6045x over previous
import jax
import jax.numpy as jnp
from jax import lax
from jax.experimental import pallas as pl
from jax.experimental.pallas import tpu as pltpu

N_DEV = 8
B_LOC = 2
SQ = 128
SKV = 128
HQ = 32
DH = 64
H_LOC = HQ // N_DEV
D_MODEL = 512
HD_LOC = H_LOC * DH
SBLK = H_LOC * SKV
VAUG = HD_LOC + 128

CW_HOPS = 4
CCW_HOPS = 3

LOCAL = 7


def kernel(x, Wq, K_ext, V_ext, Wo):
    my = lax.axis_index("i")

    kf = lax.dynamic_slice_in_dim(K_ext, my * B_LOC, B_LOC, axis=0)
    kf = kf.reshape(B_LOC, SKV, HQ * DH).astype(jnp.bfloat16)
    vf = lax.dynamic_slice_in_dim(V_ext, my * B_LOC, B_LOC, axis=0)
    vf = vf.reshape(B_LOC, SKV, HQ * DH).astype(jnp.bfloat16)

    def body(x_ref, wq_ref, wo_ref, k_ref, v_ref, out_ref,
             g_qwq, g_sq, g_qwo, g_so, xv, wqv, wov,
             xb, ctx_scr, kblkT, vblk,
             sem_in,
             s_qwq, r_qwq, s_sq, r_sq, s_qwo, r_qwo, s_so, r_so):
        my_pos = lax.axis_index("i")
        left = lax.rem(my_pos + N_DEV - 1, N_DEV)
        right = lax.rem(my_pos + 1, N_DEV)

        barrier = pltpu.get_barrier_semaphore()
        for nbr in (left, right):
            pl.semaphore_signal(barrier, inc=1, device_id=(nbr,),
                                device_id_type=pl.DeviceIdType.MESH)
        pl.semaphore_wait(barrier, 2)

        cx = pltpu.make_async_copy(x_ref, xv, sem_in.at[0])
        cwq = pltpu.make_async_copy(wq_ref, wqv, sem_in.at[1])
        cwo = pltpu.make_async_copy(wo_ref, wov, sem_in.at[2])
        for c in (cx, cwq, cwo):
            c.start()

        cwq.wait()
        wq = wqv[...]
        aq = jnp.maximum(jnp.max(jnp.abs(wq), axis=0, keepdims=True), 1e-30)
        g_qwq[LOCAL] = jnp.round(wq * (127.0 / aq)).astype(jnp.int8)
        g_sq[LOCAL] = aq * (0.125 / 127.0)
        cwo.wait()
        wo = wov[...]
        ao = jnp.maximum(jnp.max(jnp.abs(wo), axis=0, keepdims=True), 1e-30)
        g_qwo[LOCAL] = jnp.round(wo * (127.0 / ao)).astype(jnp.int8)
        g_so[LOCAL] = ao * (1.0 / 127.0)

        def chunk_rdmas(src_slot, dst_slot, sem_idx, target):
            descs = []
            for g, ssem, rsem in ((g_qwq, s_qwq, r_qwq),
                                  (g_sq, s_sq, r_sq),
                                  (g_qwo, s_qwo, r_qwo),
                                  (g_so, s_so, r_so)):
                descs.append(pltpu.make_async_remote_copy(
                    src_ref=g.at[src_slot], dst_ref=g.at[dst_slot],
                    send_sem=ssem.at[sem_idx], recv_sem=rsem.at[dst_slot],
                    device_id=(target,), device_id_type=pl.DeviceIdType.MESH))
            return descs

        def send_chunk(src_slot, dst_slot, sem_idx, target):
            descs = chunk_rdmas(src_slot, dst_slot, sem_idx, target)
            for d_ in descs:
                d_.start()
            return descs

        def wait_recv(slot):
            for d_ in chunk_rdmas(LOCAL, slot, 0, left):
                d_.wait_recv()

        sends = []
        sends += send_chunk(LOCAL, 0, 0, right)
        sends += send_chunk(LOCAL, 6, 4, left)

        for b in range(B_LOC):
            kblkT[b] = jnp.zeros((SBLK, HD_LOC), jnp.bfloat16)
            rows = lax.broadcasted_iota(jnp.int32, (SBLK, 128), 0) // SKV
            cols = lax.broadcasted_iota(jnp.int32, (SBLK, 128), 1)
            ones_pat = (rows == cols).astype(jnp.bfloat16)
            vblk[b] = jnp.concatenate(
                [jnp.zeros((SBLK, HD_LOC), jnp.bfloat16), ones_pat], axis=1)

        cx.wait()
        for b in range(B_LOC):
            xb[b * SQ:(b + 1) * SQ, :] = xv[b].astype(jnp.bfloat16)

        qblk = lax.broadcasted_iota(jnp.int32, (SQ, SBLK), 0) // 64
        kblk_id = (lax.broadcasted_iota(jnp.int32, (SQ, SBLK), 1) % SKV) // 64
        mask = kblk_id <= qblk

        def compute(d, slot, first):
            grp = lax.rem(my_pos - d + N_DEV, N_DEV)
            q_all = (jax.lax.dot_general(
                xb[...], g_qwq[slot].astype(jnp.bfloat16),
                (((1,), (0,)), ((), ())),
                preferred_element_type=jnp.float32,
            ) * g_sq[slot]).astype(jnp.bfloat16)
            for b in range(B_LOC):
                kg = k_ref[b, :, pl.ds(grp * HD_LOC, HD_LOC)]
                vg = v_ref[b, :, pl.ds(grp * HD_LOC, HD_LOC)]
                for hh in range(H_LOC):
                    kblkT[b, hh * SKV:(hh + 1) * SKV,
                          hh * DH:(hh + 1) * DH] = kg[:, hh * DH:(hh + 1) * DH]
                    vblk[b, hh * SKV:(hh + 1) * SKV,
                         hh * DH:(hh + 1) * DH] = vg[:, hh * DH:(hh + 1) * DH]
            for b in range(B_LOC):
                s = jax.lax.dot_general(
                    q_all[b * SQ:(b + 1) * SQ, :], kblkT[b],
                    (((1,), (1,)), ((), ())),
                    preferred_element_type=jnp.float32)
                w = jnp.where(mask, jnp.exp(s), 0.0).astype(jnp.bfloat16)
                aug = jax.lax.dot_general(
                    w, vblk[b],
                    (((1,), (0,)), ((), ())),
                    preferred_element_type=jnp.float32)
                rec = 1.0 / aug[:, HD_LOC:HD_LOC + H_LOC]
                scale = jnp.broadcast_to(
                    rec[:, :, None], (SQ, H_LOC, DH)).reshape(SQ, HD_LOC)
                ctx_scr[b * SQ:(b + 1) * SQ, :] = (
                    aug[:, :HD_LOC] * scale).astype(jnp.bfloat16)
            contrib = jax.lax.dot_general(
                ctx_scr[...], g_qwo[slot].astype(jnp.bfloat16),
                (((1,), (0,)), ((), ())),
                preferred_element_type=jnp.float32) * g_so[slot]
            if first:
                out_ref[...] = contrib
            else:
                out_ref[...] = out_ref[...] + contrib

        compute(0, LOCAL, first=True)

        for r in range(1, 4):
            cw = r - 1
            ccw = 7 - r
            wait_recv(cw)
            if r < CW_HOPS:
                sends += send_chunk(cw, cw + 1, r, right)
            wait_recv(ccw)
            if r < CCW_HOPS:
                sends += send_chunk(ccw, ccw - 1, 4 + r, left)
            compute(r, cw, first=False)
            compute(8 - r, ccw, first=False)

        wait_recv(3)
        compute(4, 3, first=False)

        for s_ in sends:
            s_.wait_send()

    out = pl.pallas_call(
        body,
        out_shape=jax.ShapeDtypeStruct((B_LOC * SQ, D_MODEL), jnp.float32),
        in_specs=[pl.BlockSpec(memory_space=pl.ANY)] * 3
        + [pl.BlockSpec(memory_space=pltpu.VMEM)] * 2,
        out_specs=pl.BlockSpec(memory_space=pltpu.VMEM),
        scratch_shapes=[
            pltpu.VMEM((8, D_MODEL, HD_LOC), jnp.int8),
            pltpu.VMEM((8, 1, HD_LOC), jnp.float32),
            pltpu.VMEM((8, HD_LOC, D_MODEL), jnp.int8),
            pltpu.VMEM((8, 1, D_MODEL), jnp.float32),
            pltpu.VMEM((B_LOC, SQ, D_MODEL), jnp.float32),
            pltpu.VMEM((D_MODEL, HD_LOC), jnp.float32),
            pltpu.VMEM((HD_LOC, D_MODEL), jnp.float32),
            pltpu.VMEM((B_LOC * SQ, D_MODEL), jnp.bfloat16),
            pltpu.VMEM((B_LOC * SQ, HD_LOC), jnp.bfloat16),
            pltpu.VMEM((B_LOC, SBLK, HD_LOC), jnp.bfloat16),
            pltpu.VMEM((B_LOC, SBLK, VAUG), jnp.bfloat16),
            pltpu.SemaphoreType.DMA((3,)),
            pltpu.SemaphoreType.DMA((8,)),
            pltpu.SemaphoreType.DMA((8,)),
            pltpu.SemaphoreType.DMA((8,)),
            pltpu.SemaphoreType.DMA((8,)),
            pltpu.SemaphoreType.DMA((8,)),
            pltpu.SemaphoreType.DMA((8,)),
            pltpu.SemaphoreType.DMA((8,)),
            pltpu.SemaphoreType.DMA((8,)),
        ],
        compiler_params=pltpu.CompilerParams(collective_id=0),
    )(x, Wq, Wo, kf, vf)

    return out.reshape(B_LOC, SQ, D_MODEL)


# device time: 31338 ns/iter; 2.6711x vs baseline; 1.0256x over previous
import jax
import jax.numpy as jnp
from jax import lax
from jax.experimental import pallas as pl
from jax.experimental.pallas import tpu as pltpu

N_DEV = 8
B_LOC = 2
SQ = 128
SKV = 128
HQ = 32
DH = 64
H_LOC = HQ // N_DEV
D_MODEL = 512
HD_LOC = H_LOC * DH
SBLK = H_LOC * SKV
VAUG = HD_LOC + 128

CW_HOPS = 4
CCW_HOPS = 3

LOCAL = 7


def kernel(x, Wq, K_ext, V_ext, Wo):
    my = lax.axis_index("i")

    kf = lax.dynamic_slice_in_dim(K_ext, my * B_LOC, B_LOC, axis=0)
    kf = kf.reshape(B_LOC, SKV, HQ * DH).astype(jnp.bfloat16)
    vf = lax.dynamic_slice_in_dim(V_ext, my * B_LOC, B_LOC, axis=0)
    vf = vf.reshape(B_LOC, SKV, HQ * DH).astype(jnp.bfloat16)

    def body(x_ref, wq_ref, wo_ref, k_ref, v_ref, out_ref,
             g_qwq, g_sq, g_qwo, g_so, xv, wqv, wov,
             xb, ctx_scr, kblkT, vblk, outv,
             sem_in,
             s_qwq, r_qwq, s_sq, r_sq, s_qwo, r_qwo, s_so, r_so):
        my_pos = lax.axis_index("i")
        left = lax.rem(my_pos + N_DEV - 1, N_DEV)
        right = lax.rem(my_pos + 1, N_DEV)

        cx = pltpu.make_async_copy(x_ref, xv, sem_in.at[0])
        cwq = pltpu.make_async_copy(wq_ref, wqv, sem_in.at[1])
        cwo = pltpu.make_async_copy(wo_ref, wov, sem_in.at[2])
        for c in (cx, cwq, cwo):
            c.start()

        barrier = pltpu.get_barrier_semaphore()
        for nbr in (left, right):
            pl.semaphore_signal(barrier, inc=1, device_id=(nbr,),
                                device_id_type=pl.DeviceIdType.MESH)

        cwq.wait()
        wq = wqv[...]
        aq = jnp.maximum(jnp.max(jnp.abs(wq), axis=0, keepdims=True), 1e-30)
        g_qwq[LOCAL] = jnp.round(wq * (127.0 / aq)).astype(jnp.int8)
        g_sq[LOCAL] = aq * (0.125 / 127.0)
        cwo.wait()
        wo = wov[...]
        ao = jnp.maximum(jnp.max(jnp.abs(wo), axis=0, keepdims=True), 1e-30)
        g_qwo[LOCAL] = jnp.round(wo * (127.0 / ao)).astype(jnp.int8)
        g_so[LOCAL] = ao * (1.0 / 127.0)

        pl.semaphore_wait(barrier, 2)

        def chunk_rdmas(src_slot, dst_slot, sem_idx, target):
            descs = []
            for g, ssem, rsem in ((g_qwq, s_qwq, r_qwq),
                                  (g_sq, s_sq, r_sq),
                                  (g_qwo, s_qwo, r_qwo),
                                  (g_so, s_so, r_so)):
                descs.append(pltpu.make_async_remote_copy(
                    src_ref=g.at[src_slot], dst_ref=g.at[dst_slot],
                    send_sem=ssem.at[sem_idx], recv_sem=rsem.at[dst_slot],
                    device_id=(target,), device_id_type=pl.DeviceIdType.MESH))
            return descs

        def send_chunk(src_slot, dst_slot, sem_idx, target):
            descs = chunk_rdmas(src_slot, dst_slot, sem_idx, target)
            for d_ in descs:
                d_.start()
            return descs

        def wait_recv(slot):
            for d_ in chunk_rdmas(LOCAL, slot, 0, left):
                d_.wait_recv()

        sends = []
        sends += send_chunk(LOCAL, 0, 0, right)
        sends += send_chunk(LOCAL, 6, 4, left)

        for b in range(B_LOC):
            kblkT[b] = jnp.zeros((SBLK, HD_LOC), jnp.bfloat16)
            rows = lax.broadcasted_iota(jnp.int32, (SBLK, 128), 0) // SKV
            cols = lax.broadcasted_iota(jnp.int32, (SBLK, 128), 1)
            ones_pat = (rows == cols).astype(jnp.bfloat16)
            vblk[b] = jnp.concatenate(
                [jnp.zeros((SBLK, HD_LOC), jnp.bfloat16), ones_pat], axis=1)

        cx.wait()
        for b in range(B_LOC):
            xb[b * SQ:(b + 1) * SQ, :] = xv[b].astype(jnp.bfloat16)

        qblk = lax.broadcasted_iota(jnp.int32, (SQ, SBLK), 0) // 64
        kblk_id = (lax.broadcasted_iota(jnp.int32, (SQ, SBLK), 1) % SKV) // 64
        mask = kblk_id <= qblk

        def compute(d, slot, first):
            grp = lax.rem(my_pos - d + N_DEV, N_DEV)
            q_all = (jax.lax.dot_general(
                xb[...], g_qwq[slot].astype(jnp.bfloat16),
                (((1,), (0,)), ((), ())),
                preferred_element_type=jnp.float32,
            ) * g_sq[slot]).astype(jnp.bfloat16)
            for b in range(B_LOC):
                kg = k_ref[b, :, pl.ds(grp * HD_LOC, HD_LOC)]
                vg = v_ref[b, :, pl.ds(grp * HD_LOC, HD_LOC)]
                for hh in range(H_LOC):
                    kblkT[b, hh * SKV:(hh + 1) * SKV,
                          hh * DH:(hh + 1) * DH] = kg[:, hh * DH:(hh + 1) * DH]
                    vblk[b, hh * SKV:(hh + 1) * SKV,
                         hh * DH:(hh + 1) * DH] = vg[:, hh * DH:(hh + 1) * DH]
            for b in range(B_LOC):
                s = jax.lax.dot_general(
                    q_all[b * SQ:(b + 1) * SQ, :], kblkT[b],
                    (((1,), (1,)), ((), ())),
                    preferred_element_type=jnp.float32)
                w = jnp.where(mask, jnp.exp(s), 0.0).astype(jnp.bfloat16)
                aug = jax.lax.dot_general(
                    w, vblk[b],
                    (((1,), (0,)), ((), ())),
                    preferred_element_type=jnp.float32)
                rec = 1.0 / aug[:, HD_LOC:HD_LOC + H_LOC]
                scale = jnp.broadcast_to(
                    rec[:, :, None], (SQ, H_LOC, DH)).reshape(SQ, HD_LOC)
                ctx_scr[b * SQ:(b + 1) * SQ, :] = (
                    aug[:, :HD_LOC] * scale).astype(jnp.bfloat16)
            contrib = jax.lax.dot_general(
                ctx_scr[...], g_qwo[slot].astype(jnp.bfloat16),
                (((1,), (0,)), ((), ())),
                preferred_element_type=jnp.float32) * g_so[slot]
            if first:
                outv[...] = contrib
            else:
                outv[...] = outv[...] + contrib

        compute(0, LOCAL, first=True)

        for r in range(1, 4):
            cw = r - 1
            ccw = 7 - r
            rx_cw = chunk_rdmas(LOCAL, cw, 0, left)
            fwd_cw = (chunk_rdmas(cw, cw + 1, r, right)
                      if r < CW_HOPS else None)
            for i, d_ in enumerate(rx_cw):
                d_.wait_recv()
                if fwd_cw is not None:
                    fwd_cw[i].start()
            if fwd_cw is not None:
                sends += fwd_cw
            rx_ccw = chunk_rdmas(LOCAL, ccw, 0, left)
            fwd_ccw = (chunk_rdmas(ccw, ccw - 1, 4 + r, left)
                       if r < CCW_HOPS else None)
            for i, d_ in enumerate(rx_ccw):
                d_.wait_recv()
                if fwd_ccw is not None:
                    fwd_ccw[i].start()
            if fwd_ccw is not None:
                sends += fwd_ccw
            compute(r, cw, first=False)
            compute(8 - r, ccw, first=False)

        wait_recv(3)
        compute(4, 3, first=False)

        out_ref[...] = outv[...]

        for s_ in sends:
            s_.wait_send()

    out = pl.pallas_call(
        body,
        out_shape=jax.ShapeDtypeStruct((B_LOC * SQ, D_MODEL), jnp.float32),
        in_specs=[pl.BlockSpec(memory_space=pl.ANY)] * 3
        + [pl.BlockSpec(memory_space=pltpu.VMEM)] * 2,
        out_specs=pl.BlockSpec(memory_space=pltpu.VMEM),
        scratch_shapes=[
            pltpu.VMEM((8, D_MODEL, HD_LOC), jnp.int8),
            pltpu.VMEM((8, 1, HD_LOC), jnp.float32),
            pltpu.VMEM((8, HD_LOC, D_MODEL), jnp.int8),
            pltpu.VMEM((8, 1, D_MODEL), jnp.float32),
            pltpu.VMEM((B_LOC, SQ, D_MODEL), jnp.float32),
            pltpu.VMEM((D_MODEL, HD_LOC), jnp.float32),
            pltpu.VMEM((HD_LOC, D_MODEL), jnp.float32),
            pltpu.VMEM((B_LOC * SQ, D_MODEL), jnp.bfloat16),
            pltpu.VMEM((B_LOC * SQ, HD_LOC), jnp.bfloat16),
            pltpu.VMEM((B_LOC, SBLK, HD_LOC), jnp.bfloat16),
            pltpu.VMEM((B_LOC, SBLK, VAUG), jnp.bfloat16),
            pltpu.VMEM((B_LOC * SQ, D_MODEL), jnp.float32),
            pltpu.SemaphoreType.DMA((3,)),
            pltpu.SemaphoreType.DMA((8,)),
            pltpu.SemaphoreType.DMA((8,)),
            pltpu.SemaphoreType.DMA((8,)),
            pltpu.SemaphoreType.DMA((8,)),
            pltpu.SemaphoreType.DMA((8,)),
            pltpu.SemaphoreType.DMA((8,)),
            pltpu.SemaphoreType.DMA((8,)),
            pltpu.SemaphoreType.DMA((8,)),
        ],
        compiler_params=pltpu.CompilerParams(collective_id=0),
    )(x, Wq, Wo, kf, vf)

    return out.reshape(B_LOC, SQ, D_MODEL)
